# baseline (speedup 1.0000x reference)
import functools

import jax
import jax.numpy as jnp
from jax import lax
from jax.experimental import pallas as pl
from jax.experimental.pallas import tpu as pltpu
from jax.experimental.pallas import tpu_sc as plsc

N_ROWS = 100000
N_FEAT = 128
N_SEG = 1024

NC = 2
NS = 16
L = 16

ROWS_PER_TILE = 3128
CHUNK = 128
NCHUNKS = 25
ACC_ROWS = N_SEG + 16
ROWS_PER_SUB = N_SEG // NS
DUMMY = N_SEG
CNT_W = 128


def _sc_body(feat_hbm, ids_hbm, sums_out, cnts_out,
             fbuf0, fbuf1, idbuf0, idbuf1, obuf, zbuf, zcnt, acc, cnt,
             sem0, sem1, semo0, semo1):
  c = lax.axis_index("c")
  s = lax.axis_index("s")
  g = c * NS + s
  fbufs, idbufs, sems = (fbuf0, fbuf1), (idbuf0, idbuf1), (sem0, sem1)
  semos = (semo0, semo1)

  def _init_ones(r, _):
    for k in range(CNT_W // L):
      obuf[r, pl.ds(k * L, L)] = jnp.ones((L,), jnp.float32)
    return 0
  lax.fori_loop(0, CHUNK, _init_ones, 0)

  def _init_zero(r, _):
    for k in range(N_FEAT // L):
      zbuf[r, pl.ds(k * L, L)] = jnp.zeros((L,), jnp.float32)
    for k in range(CNT_W // L):
      zcnt[r, pl.ds(k * L, L)] = jnp.zeros((L,), jnp.float32)
    return 0
  lax.fori_loop(0, ROWS_PER_SUB, _init_zero, 0)

  arow = s * ROWS_PER_SUB
  pltpu.sync_copy(zbuf, acc.at[pl.ds(arow, ROWS_PER_SUB)])
  pltpu.sync_copy(zcnt, cnt.at[pl.ds(arow, ROWS_PER_SUB)])
  plsc.subcore_barrier()

  base = g * ROWS_PER_TILE
  end = jnp.minimum(base + ROWS_PER_TILE, N_ROWS)

  def _issue(j, b):
    @pl.when(j >= 2)
    def _():
      pltpu.make_async_copy(obuf, cnt.at[idbufs[b]], semos[b]).wait()
    cb = jnp.minimum(base + j * CHUNK, N_ROWS - CHUNK)
    pltpu.async_copy(ids_hbm.at[pl.ds(cb, CHUNK)], idbufs[b], sems[b])
    pltpu.async_copy(feat_hbm.at[pl.ds(cb, CHUNK)], fbufs[b], sems[b])

  def _process(j, b):
    start = base + j * CHUNK
    cb = jnp.minimum(start, N_ROWS - CHUNK)
    pltpu.make_async_copy(ids_hbm.at[pl.ds(0, CHUNK)], idbufs[b], sems[b]).wait()
    pltpu.make_async_copy(feat_hbm.at[pl.ds(0, CHUNK)], fbufs[b], sems[b]).wait()
    for k in range(CHUNK // L):
      gi = cb + k * L + jnp.arange(L, dtype=jnp.int32)
      v = idbufs[b][pl.ds(k * L, L)]
      valid = (gi >= start) & (gi < end)
      idbufs[b][pl.ds(k * L, L)] = jnp.where(valid, v, DUMMY)
    pltpu.sync_copy(fbufs[b], acc.at[idbufs[b]], add=True)
    pltpu.async_copy(obuf, cnt.at[idbufs[b]], semos[b], add=True)

  _issue(0, 0)

  def _pair(jj, _):
    for b in range(2):
      j = jj * 2 + b
      _process(j, b)
      _issue(j + 1, (b + 1) % 2)
    return 0
  lax.fori_loop(0, NCHUNKS // 2, _pair, 0)
  _process(NCHUNKS - 1, (NCHUNKS - 1) % 2)

  pltpu.make_async_copy(obuf, cnt.at[idbufs[0]], semos[0]).wait()
  pltpu.make_async_copy(obuf, cnt.at[idbufs[1]], semos[1]).wait()

  plsc.subcore_barrier()

  pltpu.sync_copy(acc.at[pl.ds(arow, ROWS_PER_SUB)],
                  sums_out.at[c, pl.ds(arow, ROWS_PER_SUB)])
  pltpu.sync_copy(cnt.at[pl.ds(arow, ROWS_PER_SUB)],
                  cnts_out.at[c, pl.ds(arow, ROWS_PER_SUB)])


@jax.jit
def _sc_partials(feat, segment_ids):
  mesh = plsc.VectorSubcoreMesh(core_axis_name="c", subcore_axis_name="s")
  return pl.kernel(
      _sc_body,
      mesh=mesh,
      out_type=[
          jax.ShapeDtypeStruct((NC, N_SEG, N_FEAT), jnp.float32),
          jax.ShapeDtypeStruct((NC, N_SEG, CNT_W), jnp.float32),
      ],
      scratch_types=[
          pltpu.VMEM((CHUNK, N_FEAT), jnp.float32),
          pltpu.VMEM((CHUNK, N_FEAT), jnp.float32),
          pltpu.VMEM((CHUNK,), jnp.int32),
          pltpu.VMEM((CHUNK,), jnp.int32),
          pltpu.VMEM((CHUNK, CNT_W), jnp.float32),
          pltpu.VMEM((ROWS_PER_SUB, N_FEAT), jnp.float32),
          pltpu.VMEM((ROWS_PER_SUB, CNT_W), jnp.float32),
          pltpu.VMEM_SHARED((ACC_ROWS, N_FEAT), jnp.float32),
          pltpu.VMEM_SHARED((ACC_ROWS, CNT_W), jnp.float32),
          pltpu.SemaphoreType.DMA,
          pltpu.SemaphoreType.DMA,
          pltpu.SemaphoreType.DMA,
          pltpu.SemaphoreType.DMA,
      ],
  )(feat, segment_ids)


def _finalize_body(s_ref, c_ref, o_ref):
  sums = s_ref[0] + s_ref[1]
  cnts = c_ref[0, :, 0] + c_ref[1, :, 0]
  o_ref[...] = sums / jnp.maximum(cnts, 1.0)[:, None]


@jax.jit
def kernel(feat, segment_ids):
  sums, cnts = _sc_partials(feat, segment_ids.astype(jnp.int32))
  return pl.pallas_call(
      _finalize_body,
      out_shape=jax.ShapeDtypeStruct((N_SEG, N_FEAT), jnp.float32),
  )(sums, cnts)


# speedup vs baseline: 1.0319x; 1.0319x over previous
import functools

import jax
import jax.numpy as jnp
from jax import lax
from jax.experimental import pallas as pl
from jax.experimental.pallas import tpu as pltpu
from jax.experimental.pallas import tpu_sc as plsc

N_ROWS = 100000
N_FEAT = 128
N_SEG = 1024

NC = 2
NS = 16
L = 16

ROWS_PER_TILE = 3128
CHUNK = 128
NCHUNKS = 25
ACC_ROWS = N_SEG + 16
ROWS_PER_SUB = N_SEG // NS
DUMMY = N_SEG
CNT_W = 128


def _sc_body(feat_hbm, ids_hbm, sums_out, cnts_out,
             fbuf0, fbuf1, idbuf0, idbuf1, obuf, zbuf, zcnt, acc, cnt,
             sem0, sem1, semf0, semf1, semo0, semo1):
  c = lax.axis_index("c")
  s = lax.axis_index("s")
  g = c * NS + s
  fbufs, idbufs, sems = (fbuf0, fbuf1), (idbuf0, idbuf1), (sem0, sem1)
  semfs, semos = (semf0, semf1), (semo0, semo1)

  def _init_ones(r, _):
    for k in range(CNT_W // L):
      obuf[r, pl.ds(k * L, L)] = jnp.ones((L,), jnp.float32)
    return 0
  lax.fori_loop(0, CHUNK, _init_ones, 0)

  def _init_zero(r, _):
    for k in range(N_FEAT // L):
      zbuf[r, pl.ds(k * L, L)] = jnp.zeros((L,), jnp.float32)
    for k in range(CNT_W // L):
      zcnt[r, pl.ds(k * L, L)] = jnp.zeros((L,), jnp.float32)
    return 0
  lax.fori_loop(0, ROWS_PER_SUB, _init_zero, 0)

  arow = s * ROWS_PER_SUB
  pltpu.sync_copy(zbuf, acc.at[pl.ds(arow, ROWS_PER_SUB)])
  pltpu.sync_copy(zcnt, cnt.at[pl.ds(arow, ROWS_PER_SUB)])
  plsc.subcore_barrier()

  base = g * ROWS_PER_TILE
  end = jnp.minimum(base + ROWS_PER_TILE, N_ROWS)

  def _issue(j, b):
    @pl.when(j >= 2)
    def _():
      pltpu.make_async_copy(fbufs[b], acc.at[idbufs[b]], semfs[b]).wait()
      pltpu.make_async_copy(obuf, cnt.at[idbufs[b]], semos[b]).wait()
    cb = jnp.minimum(base + j * CHUNK, N_ROWS - CHUNK)
    pltpu.async_copy(ids_hbm.at[pl.ds(cb, CHUNK)], idbufs[b], sems[b])
    pltpu.async_copy(feat_hbm.at[pl.ds(cb, CHUNK)], fbufs[b], sems[b])

  def _process(j, b):
    start = base + j * CHUNK
    cb = jnp.minimum(start, N_ROWS - CHUNK)
    pltpu.make_async_copy(ids_hbm.at[pl.ds(0, CHUNK)], idbufs[b], sems[b]).wait()
    pltpu.make_async_copy(feat_hbm.at[pl.ds(0, CHUNK)], fbufs[b], sems[b]).wait()
    for k in range(CHUNK // L):
      gi = cb + k * L + jnp.arange(L, dtype=jnp.int32)
      v = idbufs[b][pl.ds(k * L, L)]
      valid = (gi >= start) & (gi < end)
      idbufs[b][pl.ds(k * L, L)] = jnp.where(valid, v, DUMMY)
    pltpu.async_copy(fbufs[b], acc.at[idbufs[b]], semfs[b], add=True)
    pltpu.async_copy(obuf, cnt.at[idbufs[b]], semos[b], add=True)

  _issue(0, 0)

  def _pair(jj, _):
    for b in range(2):
      j = jj * 2 + b
      _issue(j + 1, (b + 1) % 2)
      _process(j, b)
    return 0
  lax.fori_loop(0, NCHUNKS // 2, _pair, 0)
  _process(NCHUNKS - 1, (NCHUNKS - 1) % 2)

  for b in range(2):
    pltpu.make_async_copy(fbufs[b], acc.at[idbufs[b]], semfs[b]).wait()
    pltpu.make_async_copy(obuf, cnt.at[idbufs[b]], semos[b]).wait()

  plsc.subcore_barrier()

  pltpu.sync_copy(acc.at[pl.ds(arow, ROWS_PER_SUB)],
                  sums_out.at[c, pl.ds(arow, ROWS_PER_SUB)])
  pltpu.sync_copy(cnt.at[pl.ds(arow, ROWS_PER_SUB)],
                  cnts_out.at[c, pl.ds(arow, ROWS_PER_SUB)])


@jax.jit
def _sc_partials(feat, segment_ids):
  mesh = plsc.VectorSubcoreMesh(core_axis_name="c", subcore_axis_name="s")
  return pl.kernel(
      _sc_body,
      mesh=mesh,
      out_type=[
          jax.ShapeDtypeStruct((NC, N_SEG, N_FEAT), jnp.float32),
          jax.ShapeDtypeStruct((NC, N_SEG, CNT_W), jnp.float32),
      ],
      scratch_types=[
          pltpu.VMEM((CHUNK, N_FEAT), jnp.float32),
          pltpu.VMEM((CHUNK, N_FEAT), jnp.float32),
          pltpu.VMEM((CHUNK,), jnp.int32),
          pltpu.VMEM((CHUNK,), jnp.int32),
          pltpu.VMEM((CHUNK, CNT_W), jnp.float32),
          pltpu.VMEM((ROWS_PER_SUB, N_FEAT), jnp.float32),
          pltpu.VMEM((ROWS_PER_SUB, CNT_W), jnp.float32),
          pltpu.VMEM_SHARED((ACC_ROWS, N_FEAT), jnp.float32),
          pltpu.VMEM_SHARED((ACC_ROWS, CNT_W), jnp.float32),
          pltpu.SemaphoreType.DMA,
          pltpu.SemaphoreType.DMA,
          pltpu.SemaphoreType.DMA,
          pltpu.SemaphoreType.DMA,
          pltpu.SemaphoreType.DMA,
          pltpu.SemaphoreType.DMA,
      ],
  )(feat, segment_ids)


def _finalize_body(s_ref, c_ref, o_ref):
  sums = s_ref[0] + s_ref[1]
  cnts = c_ref[0, :, 0] + c_ref[1, :, 0]
  o_ref[...] = sums / jnp.maximum(cnts, 1.0)[:, None]


@jax.jit
def kernel(feat, segment_ids):
  sums, cnts = _sc_partials(feat, segment_ids.astype(jnp.int32))
  return pl.pallas_call(
      _finalize_body,
      out_shape=jax.ShapeDtypeStruct((N_SEG, N_FEAT), jnp.float32),
  )(sums, cnts)


# speedup vs baseline: 1.0503x; 1.0178x over previous
import functools

import jax
import jax.numpy as jnp
from jax import lax
from jax.experimental import pallas as pl
from jax.experimental.pallas import tpu as pltpu
from jax.experimental.pallas import tpu_sc as plsc

N_ROWS = 100000
N_FEAT = 128
N_SEG = 1024

NC = 2
NS = 16
L = 16

ROWS_PER_TILE = 3128
CHUNK = 128
NCHUNKS = 25
ACC_ROWS = N_SEG + 16
ROWS_PER_SUB = N_SEG // NS
DUMMY = N_SEG
CNT_W = 128


def _sc_body(feat_hbm, ids_hbm, sums_out, cnts_out,
             fbuf0, fbuf1, idbuf0, idbuf1, obuf, zbuf, zcnt, acc, cnt,
             sem0, sem1):
  c = lax.axis_index("c")
  s = lax.axis_index("s")
  g = c * NS + s
  fbufs, idbufs, sems = (fbuf0, fbuf1), (idbuf0, idbuf1), (sem0, sem1)

  def _init_ones(r, _):
    for k in range(CNT_W // L):
      obuf[r, pl.ds(k * L, L)] = jnp.ones((L,), jnp.float32)
    return 0
  lax.fori_loop(0, CHUNK, _init_ones, 0)

  def _init_zero(r, _):
    for k in range(N_FEAT // L):
      zbuf[r, pl.ds(k * L, L)] = jnp.zeros((L,), jnp.float32)
    for k in range(CNT_W // L):
      zcnt[r, pl.ds(k * L, L)] = jnp.zeros((L,), jnp.float32)
    return 0
  lax.fori_loop(0, ROWS_PER_SUB, _init_zero, 0)

  arow = s * ROWS_PER_SUB
  pltpu.sync_copy(zbuf, acc.at[pl.ds(arow, ROWS_PER_SUB)])
  pltpu.sync_copy(zcnt, cnt.at[pl.ds(arow, ROWS_PER_SUB)])
  plsc.subcore_barrier()

  base = g * ROWS_PER_TILE
  end = jnp.minimum(base + ROWS_PER_TILE, N_ROWS)

  def _issue(j, b):
    cb = jnp.minimum(base + j * CHUNK, N_ROWS - CHUNK)
    pltpu.async_copy(ids_hbm.at[pl.ds(cb, CHUNK)], idbufs[b], sems[b])
    pltpu.async_copy(feat_hbm.at[pl.ds(cb, CHUNK)], fbufs[b], sems[b])

  def _process(j, b):
    start = base + j * CHUNK
    cb = jnp.minimum(start, N_ROWS - CHUNK)
    pltpu.make_async_copy(ids_hbm.at[pl.ds(0, CHUNK)], idbufs[b], sems[b]).wait()
    pltpu.make_async_copy(feat_hbm.at[pl.ds(0, CHUNK)], fbufs[b], sems[b]).wait()
    for k in range(CHUNK // L):
      gi = cb + k * L + jnp.arange(L, dtype=jnp.int32)
      v = idbufs[b][pl.ds(k * L, L)]
      valid = (gi >= start) & (gi < end)
      idbufs[b][pl.ds(k * L, L)] = jnp.where(valid, v, DUMMY)
    pltpu.sync_copy(fbufs[b], acc.at[idbufs[b]], add=True)
    pltpu.sync_copy(obuf, cnt.at[idbufs[b]], add=True)

  _issue(0, 0)

  def _pair(jj, _):
    for b in range(2):
      j = jj * 2 + b
      _issue(j + 1, (b + 1) % 2)
      _process(j, b)
    return 0
  lax.fori_loop(0, NCHUNKS // 2, _pair, 0)
  _process(NCHUNKS - 1, (NCHUNKS - 1) % 2)

  plsc.subcore_barrier()

  pltpu.sync_copy(acc.at[pl.ds(arow, ROWS_PER_SUB)],
                  sums_out.at[c, pl.ds(arow, ROWS_PER_SUB)])
  pltpu.sync_copy(cnt.at[pl.ds(arow, ROWS_PER_SUB)],
                  cnts_out.at[c, pl.ds(arow, ROWS_PER_SUB)])


@jax.jit
def _sc_partials(feat, segment_ids):
  mesh = plsc.VectorSubcoreMesh(core_axis_name="c", subcore_axis_name="s")
  return pl.kernel(
      _sc_body,
      mesh=mesh,
      out_type=[
          jax.ShapeDtypeStruct((NC, N_SEG, N_FEAT), jnp.float32),
          jax.ShapeDtypeStruct((NC, N_SEG, CNT_W), jnp.float32),
      ],
      scratch_types=[
          pltpu.VMEM((CHUNK, N_FEAT), jnp.float32),
          pltpu.VMEM((CHUNK, N_FEAT), jnp.float32),
          pltpu.VMEM((CHUNK,), jnp.int32),
          pltpu.VMEM((CHUNK,), jnp.int32),
          pltpu.VMEM((CHUNK, CNT_W), jnp.float32),
          pltpu.VMEM((ROWS_PER_SUB, N_FEAT), jnp.float32),
          pltpu.VMEM((ROWS_PER_SUB, CNT_W), jnp.float32),
          pltpu.VMEM_SHARED((ACC_ROWS, N_FEAT), jnp.float32),
          pltpu.VMEM_SHARED((ACC_ROWS, CNT_W), jnp.float32),
          pltpu.SemaphoreType.DMA,
          pltpu.SemaphoreType.DMA,
      ],
  )(feat, segment_ids)


def _finalize_body(s_ref, c_ref, o_ref):
  sums = s_ref[0] + s_ref[1]
  cnts = c_ref[0, :, 0] + c_ref[1, :, 0]
  o_ref[...] = sums / jnp.maximum(cnts, 1.0)[:, None]


@jax.jit
def kernel(feat, segment_ids):
  sums, cnts = _sc_partials(feat, segment_ids.astype(jnp.int32))
  return pl.pallas_call(
      _finalize_body,
      out_shape=jax.ShapeDtypeStruct((N_SEG, N_FEAT), jnp.float32),
  )(sums, cnts)
